# Initial kernel scaffold; baseline (speedup 1.0000x reference)
import jax
import jax.numpy as jnp
from jax import lax
from jax.experimental import pallas as pl
from jax.experimental.pallas import tpu as pltpu
from jax.experimental.pallas import tpu_sc as plsc

BATCH = 16384
N_FIELDS = 26
NC = 2
NS = 16
NW = NC * NS
ROWS_PER_W = BATCH // NW
VALS_PER_W = ROWS_PER_W * N_FIELDS
CHUNK = 128
N_CHUNKS = VALS_PER_W // CHUNK
N_GROUPS = ROWS_PER_W // 16


def _sc_body(idx_hbm, w_hbm, b_hbm, out_hbm, idx_v, vals_v, out_v, b_v, sem):
    wid = lax.axis_index("s") * NC + lax.axis_index("c")

    pltpu.sync_copy(idx_hbm.at[wid], idx_v)
    pltpu.sync_copy(b_hbm, b_v)

    def fire(c, carry):
        pltpu.make_async_copy(
            w_hbm.at[idx_v.at[c]], vals_v.at[pl.ds(c * CHUNK, CHUNK)], sem
        ).start()
        return carry

    lax.fori_loop(0, N_CHUNKS, fire, 0)
    pltpu.make_async_copy(w_hbm.at[pl.ds(0, VALS_PER_W)], vals_v, sem).wait()

    lane = lax.iota(jnp.int32, 16)
    bias = b_v[...]

    def group(g, carry):
        base = (pl.multiple_of(g * 16, 16) + lane) * N_FIELDS
        acc = bias
        for f in range(N_FIELDS):
            acc = acc + plsc.load_gather(vals_v, [base + f])
        y = 1.0 / (1.0 + jnp.exp(-acc))
        out_v[pl.ds(pl.multiple_of(g * 16, 16), 16)] = y
        return carry

    lax.fori_loop(0, N_GROUPS, group, 0)
    pltpu.sync_copy(out_v, out_hbm.at[pl.ds(wid * ROWS_PER_W, ROWS_PER_W)])


@jax.jit
def kernel(indices, w, b):
    idx = indices.astype(jnp.int32).reshape(NW, N_CHUNKS, CHUNK)
    wf = w.reshape(-1)
    b16 = jnp.broadcast_to(b.astype(jnp.float32), (16,))

    run = pl.kernel(
        _sc_body,
        out_type=jax.ShapeDtypeStruct((BATCH,), jnp.float32),
        mesh=plsc.VectorSubcoreMesh(core_axis_name="c", subcore_axis_name="s"),
        scratch_types=[
            pltpu.VMEM((N_CHUNKS, CHUNK), jnp.int32),
            pltpu.VMEM((VALS_PER_W,), jnp.float32),
            pltpu.VMEM((ROWS_PER_W,), jnp.float32),
            pltpu.VMEM((16,), jnp.float32),
            pltpu.SemaphoreType.DMA,
        ],
    )
    return run(idx, wf, b16)


# speedup vs baseline: 1.2875x; 1.2875x over previous
import jax
import jax.numpy as jnp
from jax import lax
from jax.experimental import pallas as pl
from jax.experimental.pallas import tpu as pltpu
from jax.experimental.pallas import tpu_sc as plsc

BATCH = 16384
N_FIELDS = 26
NC = 2
NS = 16
NW = NC * NS
ROWS_PER_W = BATCH // NW
VALS_PER_W = ROWS_PER_W * N_FIELDS
CHUNK = 128
N_CHUNKS = VALS_PER_W // CHUNK
N_GROUPS = ROWS_PER_W // 16


def _sc_body(idx_hbm, w_hbm, b_hbm, out_hbm, idx_v, vals_v, out_v, b_v, sem):
    wid = lax.axis_index("s") * NC + lax.axis_index("c")

    pltpu.sync_copy(idx_hbm.at[wid], idx_v)
    pltpu.sync_copy(b_hbm, b_v)

    def fire(c, carry):
        pltpu.make_async_copy(
            w_hbm.at[idx_v.at[c]], vals_v.at[pl.ds(c * CHUNK, CHUNK)], sem
        ).start()
        return carry

    lax.fori_loop(0, N_CHUNKS, fire, 0)
    pltpu.make_async_copy(w_hbm.at[pl.ds(0, VALS_PER_W)], vals_v, sem).wait()

    lane = lax.iota(jnp.int32, 16)
    bias = b_v[...]

    def group(g, carry):
        base = (pl.multiple_of(g * 16, 16) + lane) * N_FIELDS
        acc = bias
        for f in range(N_FIELDS):
            acc = acc + plsc.load_gather(vals_v, [base + f])
        y = 1.0 / (1.0 + jnp.exp(-acc))
        out_v[pl.ds(pl.multiple_of(g * 16, 16), 16)] = y
        return carry

    lax.fori_loop(0, N_GROUPS, group, 0)
    pltpu.sync_copy(out_v, out_hbm.at[pl.ds(wid * ROWS_PER_W, ROWS_PER_W)])


@jax.jit
def kernel(indices, w, b):
    idx = indices.astype(jnp.int32).reshape(NW, N_CHUNKS, CHUNK)
    wf = w.reshape(-1)
    b16 = jnp.broadcast_to(b.astype(jnp.float32), (16,))

    run = pl.kernel(
        _sc_body,
        out_type=jax.ShapeDtypeStruct((BATCH,), jnp.float32),
        mesh=plsc.VectorSubcoreMesh(core_axis_name="c", subcore_axis_name="s"),
        compiler_params=pltpu.CompilerParams(needs_layout_passes=False),
        scratch_types=[
            pltpu.VMEM((N_CHUNKS, CHUNK), jnp.int32),
            pltpu.VMEM((VALS_PER_W,), jnp.float32),
            pltpu.VMEM((ROWS_PER_W,), jnp.float32),
            pltpu.VMEM((16,), jnp.float32),
            pltpu.SemaphoreType.DMA,
        ],
    )
    return run(idx, wf, b16)


# speedup vs baseline: 1.3084x; 1.0162x over previous
import jax
import jax.numpy as jnp
from jax import lax
from jax.experimental import pallas as pl
from jax.experimental.pallas import tpu as pltpu
from jax.experimental.pallas import tpu_sc as plsc

BATCH = 16384
N_FIELDS = 26
NC = 2
NS = 16
NW = NC * NS
ROWS_PER_W = BATCH // NW
VALS_PER_W = ROWS_PER_W * N_FIELDS
CHUNK = 128
N_CHUNKS = VALS_PER_W // CHUNK
N_GROUPS = ROWS_PER_W // 16


def _sc_body(idx_hbm, w_hbm, b_hbm, out_hbm, idx_v, vals_v, out_v, b_v, sem):
    wid = lax.axis_index("s") * NC + lax.axis_index("c")

    pltpu.sync_copy(idx_hbm.at[wid], idx_v)
    pltpu.sync_copy(b_hbm, b_v)

    pltpu.make_async_copy(w_hbm.at[idx_v], vals_v, sem).start()
    pltpu.make_async_copy(w_hbm.at[pl.ds(0, VALS_PER_W)], vals_v, sem).wait()

    lane = lax.iota(jnp.int32, 16)
    bias = b_v[...]

    def group(g, carry):
        base = (pl.multiple_of(g * 16, 16) + lane) * N_FIELDS
        acc = bias
        for f in range(N_FIELDS):
            acc = acc + plsc.load_gather(vals_v, [base + f])
        y = 1.0 / (1.0 + jnp.exp(-acc))
        out_v[pl.ds(pl.multiple_of(g * 16, 16), 16)] = y
        return carry

    lax.fori_loop(0, N_GROUPS, group, 0)
    pltpu.sync_copy(out_v, out_hbm.at[pl.ds(wid * ROWS_PER_W, ROWS_PER_W)])


@jax.jit
def kernel(indices, w, b):
    idx = indices.astype(jnp.int32).reshape(NW, VALS_PER_W)
    wf = w.reshape(-1)
    b16 = jnp.broadcast_to(b.astype(jnp.float32), (16,))

    run = pl.kernel(
        _sc_body,
        out_type=jax.ShapeDtypeStruct((BATCH,), jnp.float32),
        mesh=plsc.VectorSubcoreMesh(core_axis_name="c", subcore_axis_name="s"),
        compiler_params=pltpu.CompilerParams(needs_layout_passes=False),
        scratch_types=[
            pltpu.VMEM((VALS_PER_W,), jnp.int32),
            pltpu.VMEM((VALS_PER_W,), jnp.float32),
            pltpu.VMEM((ROWS_PER_W,), jnp.float32),
            pltpu.VMEM((16,), jnp.float32),
            pltpu.SemaphoreType.DMA,
        ],
    )
    return run(idx, wf, b16)


# speedup vs baseline: 1.5247x; 1.1654x over previous
import jax
import jax.numpy as jnp
from jax import lax
from jax.experimental import pallas as pl
from jax.experimental.pallas import tpu as pltpu
from jax.experimental.pallas import tpu_sc as plsc

BATCH = 16384
N_FIELDS = 26
INPUT_DIM = 1000000
NC = 2
NS = 16
NW = NC * NS
ROWS_PER_W = BATCH // NW
VALS_PER_W = ROWS_PER_W * N_FIELDS
N_GROUPS = ROWS_PER_W // 16


def _sc_body(idx_hbm, w_hbm, b_hbm, out_hbm, idx_v, vals_v, out_v, b_v, sem):
    wid = lax.axis_index("s") * NC + lax.axis_index("c")
    base = wid * ROWS_PER_W

    for f in range(N_FIELDS):
        pltpu.make_async_copy(
            idx_hbm.at[f, pl.ds(base, ROWS_PER_W)],
            idx_v.at[pl.ds(f * ROWS_PER_W, ROWS_PER_W)],
            sem,
        ).start()
    pltpu.sync_copy(b_hbm, b_v)
    pltpu.make_async_copy(idx_hbm.at[0, pl.ds(0, VALS_PER_W)], idx_v, sem).wait()

    for f in range(N_FIELDS):
        pltpu.make_async_copy(
            w_hbm.at[idx_v.at[pl.ds(f * ROWS_PER_W, ROWS_PER_W)]],
            vals_v.at[pl.ds(f * ROWS_PER_W, ROWS_PER_W)],
            sem,
        ).start()
    pltpu.make_async_copy(w_hbm.at[pl.ds(0, VALS_PER_W)], vals_v, sem).wait()

    bias = b_v[...]

    def group(g, carry):
        off = pl.multiple_of(g * 16, 16)
        acc = bias
        for f in range(N_FIELDS):
            acc = acc + vals_v[pl.ds(f * ROWS_PER_W + off, 16)]
        y = 1.0 / (1.0 + jnp.exp(-acc))
        out_v[pl.ds(off, 16)] = y
        return carry

    lax.fori_loop(0, N_GROUPS, group, 0)
    pltpu.sync_copy(out_v, out_hbm.at[pl.ds(base, ROWS_PER_W)])


@jax.jit
def kernel(indices, w, b):
    idx_t = indices.astype(jnp.int32).T
    wf = lax.reshape(w, (INPUT_DIM,), dimensions=(1, 0))
    b16 = jnp.broadcast_to(b.astype(jnp.float32), (16,))

    run = pl.kernel(
        _sc_body,
        out_type=jax.ShapeDtypeStruct((BATCH,), jnp.float32),
        mesh=plsc.VectorSubcoreMesh(core_axis_name="c", subcore_axis_name="s"),
        compiler_params=pltpu.CompilerParams(needs_layout_passes=False),
        scratch_types=[
            pltpu.VMEM((VALS_PER_W,), jnp.int32),
            pltpu.VMEM((VALS_PER_W,), jnp.float32),
            pltpu.VMEM((ROWS_PER_W,), jnp.float32),
            pltpu.VMEM((16,), jnp.float32),
            pltpu.SemaphoreType.DMA,
        ],
    )
    return run(idx_t, wf, b16)


# speedup vs baseline: 3.1916x; 2.0932x over previous
import jax
import jax.numpy as jnp
from jax import lax
from jax.experimental import pallas as pl
from jax.experimental.pallas import tpu as pltpu
from jax.experimental.pallas import tpu_sc as plsc

BATCH = 16384
N_FIELDS = 26
INPUT_DIM = 1000000
NC = 2
NS = 16
NW = NC * NS
ROWS_PER_W = BATCH // NW
VALS_PER_W = ROWS_PER_W * N_FIELDS
N_GROUPS = ROWS_PER_W // 16


def _sc_body(idx_hbm, w_hbm, b_hbm, out_hbm, idx_v, vals_v, out_v, b_v, sem):
    wid = lax.axis_index("s") * NC + lax.axis_index("c")
    base = wid * ROWS_PER_W

    for f in range(N_FIELDS):
        pltpu.make_async_copy(
            idx_hbm.at[f, pl.ds(base, ROWS_PER_W)],
            idx_v.at[pl.ds(f * ROWS_PER_W, ROWS_PER_W)],
            sem,
        ).start()
    pltpu.sync_copy(b_hbm, b_v)
    pltpu.make_async_copy(idx_hbm.at[0, pl.ds(0, VALS_PER_W)], idx_v, sem).wait()

    wrow = w_hbm.at[0]
    for f in range(N_FIELDS):
        pltpu.make_async_copy(
            wrow.at[idx_v.at[pl.ds(f * ROWS_PER_W, ROWS_PER_W)]],
            vals_v.at[pl.ds(f * ROWS_PER_W, ROWS_PER_W)],
            sem,
        ).start()
    pltpu.make_async_copy(wrow.at[pl.ds(0, VALS_PER_W)], vals_v, sem).wait()

    bias = b_v[...]

    def group(g, carry):
        off = pl.multiple_of(g * 16, 16)
        acc = bias
        for f in range(N_FIELDS):
            acc = acc + vals_v[pl.ds(f * ROWS_PER_W + off, 16)]
        y = 1.0 / (1.0 + jnp.exp(-acc))
        out_v[pl.ds(off, 16)] = y
        return carry

    lax.fori_loop(0, N_GROUPS, group, 0)
    pltpu.sync_copy(out_v, out_hbm.at[pl.ds(base, ROWS_PER_W)])


@jax.jit
def kernel(indices, w, b):
    idx_t = indices.astype(jnp.int32).T
    wf = w.T
    b16 = jnp.broadcast_to(b.astype(jnp.float32), (16,))

    run = pl.kernel(
        _sc_body,
        out_type=jax.ShapeDtypeStruct((BATCH,), jnp.float32),
        mesh=plsc.VectorSubcoreMesh(core_axis_name="c", subcore_axis_name="s"),
        compiler_params=pltpu.CompilerParams(needs_layout_passes=False),
        scratch_types=[
            pltpu.VMEM((VALS_PER_W,), jnp.int32),
            pltpu.VMEM((VALS_PER_W,), jnp.float32),
            pltpu.VMEM((ROWS_PER_W,), jnp.float32),
            pltpu.VMEM((16,), jnp.float32),
            pltpu.SemaphoreType.DMA,
        ],
    )
    return run(idx_t, wf, b16)


# speedup vs baseline: 3.2953x; 1.0325x over previous
import jax
import jax.numpy as jnp
from jax import lax
from jax.experimental import pallas as pl
from jax.experimental.pallas import tpu as pltpu
from jax.experimental.pallas import tpu_sc as plsc

BATCH = 16384
N_FIELDS = 26
INPUT_DIM = 1000000
NC = 2
NS = 16
NW = NC * NS
ROWS_PER_W = BATCH // NW
VALS_PER_W = ROWS_PER_W * N_FIELDS
N_GROUPS = ROWS_PER_W // 16


def _sc_body(idx_hbm, w_hbm, b_hbm, out_hbm, idx_v, vals_v, acc_v, out_v, b_v,
             isem, gsem):
    wid = lax.axis_index("s") * NC + lax.axis_index("c")
    base = wid * ROWS_PER_W

    for f in range(N_FIELDS):
        pltpu.make_async_copy(
            idx_hbm.at[f, pl.ds(base, ROWS_PER_W)],
            idx_v.at[pl.ds(f * ROWS_PER_W, ROWS_PER_W)],
            isem,
        ).start()
    pltpu.sync_copy(b_hbm, b_v)
    pltpu.make_async_copy(idx_hbm.at[0, pl.ds(0, VALS_PER_W)], idx_v, isem).wait()

    wrow = w_hbm.at[0]

    def gather(f):
        return pltpu.make_async_copy(
            wrow.at[idx_v.at[pl.ds(f * ROWS_PER_W, ROWS_PER_W)]],
            vals_v.at[pl.ds(f * ROWS_PER_W, ROWS_PER_W)],
            gsem.at[f],
        )

    for f in range(N_FIELDS):
        gather(f).start()

    bias = b_v[...]

    gather(0).wait()

    def init_group(g, carry):
        off = pl.multiple_of(g * 16, 16)
        acc_v[pl.ds(off, 16)] = bias + vals_v[pl.ds(off, 16)]
        return carry

    lax.fori_loop(0, N_GROUPS, init_group, 0)

    for f in range(1, N_FIELDS):
        gather(f).wait()

        def add_group(g, carry, f=f):
            off = pl.multiple_of(g * 16, 16)
            plsc.addupdate(
                acc_v.at[pl.ds(off, 16)],
                vals_v[pl.ds(f * ROWS_PER_W + off, 16)],
            )
            return carry

        lax.fori_loop(0, N_GROUPS, add_group, 0)

    def sig_group(g, carry):
        off = pl.multiple_of(g * 16, 16)
        out_v[pl.ds(off, 16)] = 1.0 / (1.0 + jnp.exp(-acc_v[pl.ds(off, 16)]))
        return carry

    lax.fori_loop(0, N_GROUPS, sig_group, 0)
    pltpu.sync_copy(out_v, out_hbm.at[pl.ds(base, ROWS_PER_W)])


@jax.jit
def kernel(indices, w, b):
    idx_t = indices.astype(jnp.int32).T
    wf = w.T
    b16 = jnp.broadcast_to(b.astype(jnp.float32), (16,))

    run = pl.kernel(
        _sc_body,
        out_type=jax.ShapeDtypeStruct((BATCH,), jnp.float32),
        mesh=plsc.VectorSubcoreMesh(core_axis_name="c", subcore_axis_name="s"),
        compiler_params=pltpu.CompilerParams(needs_layout_passes=False),
        scratch_types=[
            pltpu.VMEM((VALS_PER_W,), jnp.int32),
            pltpu.VMEM((VALS_PER_W,), jnp.float32),
            pltpu.VMEM((ROWS_PER_W,), jnp.float32),
            pltpu.VMEM((ROWS_PER_W,), jnp.float32),
            pltpu.VMEM((16,), jnp.float32),
            pltpu.SemaphoreType.DMA,
            pltpu.SemaphoreType.DMA((N_FIELDS,)),
        ],
    )
    return run(idx_t, wf, b16)


# speedup vs baseline: 3.3017x; 1.0020x over previous
import jax
import jax.numpy as jnp
from jax import lax
from jax.experimental import pallas as pl
from jax.experimental.pallas import tpu as pltpu
from jax.experimental.pallas import tpu_sc as plsc

BATCH = 16384
N_FIELDS = 26
INPUT_DIM = 1000000
NC = 2
NS = 16
NW = NC * NS
ROWS_PER_W = BATCH // NW
VALS_PER_W = ROWS_PER_W * N_FIELDS
N_GROUPS = ROWS_PER_W // 16


def _sc_body(idx_hbm, w_hbm, b_hbm, out_hbm, idx_v, vals_v, acc_v, out_v, b_v,
             isem, gsem, bsem):
    wid = lax.axis_index("s") * NC + lax.axis_index("c")
    base = wid * ROWS_PER_W

    for f in range(N_FIELDS):
        pltpu.make_async_copy(
            idx_hbm.at[f, pl.ds(base, ROWS_PER_W)],
            idx_v.at[pl.ds(f * ROWS_PER_W, ROWS_PER_W)],
            isem,
        ).start()
    bcopy = pltpu.make_async_copy(b_hbm, b_v, bsem)
    bcopy.start()
    pltpu.make_async_copy(idx_hbm.at[0, pl.ds(0, VALS_PER_W)], idx_v, isem).wait()

    wrow = w_hbm.at[0]

    def gather(f):
        return pltpu.make_async_copy(
            wrow.at[idx_v.at[pl.ds(f * ROWS_PER_W, ROWS_PER_W)]],
            vals_v.at[pl.ds(f * ROWS_PER_W, ROWS_PER_W)],
            gsem.at[f],
        )

    for f in range(N_FIELDS):
        gather(f).start()

    bcopy.wait()
    bias = b_v[...]

    gather(0).wait()

    def init_group(g, carry):
        off = pl.multiple_of(g * 16, 16)
        acc_v[pl.ds(off, 16)] = bias + vals_v[pl.ds(off, 16)]
        return carry

    lax.fori_loop(0, N_GROUPS, init_group, 0)

    for f in range(1, N_FIELDS):
        gather(f).wait()

        def add_group(g, carry, f=f):
            off = pl.multiple_of(g * 16, 16)
            plsc.addupdate(
                acc_v.at[pl.ds(off, 16)],
                vals_v[pl.ds(f * ROWS_PER_W + off, 16)],
            )
            return carry

        lax.fori_loop(0, N_GROUPS, add_group, 0)

    def sig_group(g, carry):
        off = pl.multiple_of(g * 16, 16)
        out_v[pl.ds(off, 16)] = 1.0 / (1.0 + jnp.exp(-acc_v[pl.ds(off, 16)]))
        return carry

    lax.fori_loop(0, N_GROUPS, sig_group, 0)
    pltpu.sync_copy(out_v, out_hbm.at[pl.ds(base, ROWS_PER_W)])


@jax.jit
def kernel(indices, w, b):
    idx_t = indices.astype(jnp.int32).T
    wf = w.T
    b16 = jnp.broadcast_to(b.astype(jnp.float32), (16,))

    run = pl.kernel(
        _sc_body,
        out_type=jax.ShapeDtypeStruct((BATCH,), jnp.float32),
        mesh=plsc.VectorSubcoreMesh(core_axis_name="c", subcore_axis_name="s"),
        compiler_params=pltpu.CompilerParams(needs_layout_passes=False),
        scratch_types=[
            pltpu.VMEM((VALS_PER_W,), jnp.int32),
            pltpu.VMEM((VALS_PER_W,), jnp.float32),
            pltpu.VMEM((ROWS_PER_W,), jnp.float32),
            pltpu.VMEM((ROWS_PER_W,), jnp.float32),
            pltpu.VMEM((16,), jnp.float32),
            pltpu.SemaphoreType.DMA,
            pltpu.SemaphoreType.DMA((N_FIELDS,)),
            pltpu.SemaphoreType.DMA,
        ],
    )
    return run(idx_t, wf, b16)


# speedup vs baseline: 3.3191x; 1.0053x over previous
import jax
import jax.numpy as jnp
from jax import lax
from jax.experimental import pallas as pl
from jax.experimental.pallas import tpu as pltpu
from jax.experimental.pallas import tpu_sc as plsc

BATCH = 16384
N_FIELDS = 26
INPUT_DIM = 1000000
NC = 2
NS = 16
NW = NC * NS
ROWS_PER_W = BATCH // NW
VALS_PER_W = ROWS_PER_W * N_FIELDS
N_GROUPS = ROWS_PER_W // 16


def _sc_body(idx_hbm, w_hbm, b_hbm, out_hbm, idx_v, vals_v, acc_v, out_v, b_v,
             isem, hsem, gsem, bsem):
    wid = lax.axis_index("s") * NC + lax.axis_index("c")
    base = wid * ROWS_PER_W

    HEAD = 4

    def stage(f, sem):
        return pltpu.make_async_copy(
            idx_hbm.at[f, pl.ds(base, ROWS_PER_W)],
            idx_v.at[pl.ds(f * ROWS_PER_W, ROWS_PER_W)],
            sem,
        )

    for f in range(HEAD):
        stage(f, hsem).start()
    for f in range(HEAD, N_FIELDS):
        stage(f, isem).start()
    bcopy = pltpu.make_async_copy(b_hbm, b_v, bsem)
    bcopy.start()

    wrow = w_hbm.at[0]

    def gather(f):
        return pltpu.make_async_copy(
            wrow.at[idx_v.at[pl.ds(f * ROWS_PER_W, ROWS_PER_W)]],
            vals_v.at[pl.ds(f * ROWS_PER_W, ROWS_PER_W)],
            gsem.at[f],
        )

    pltpu.make_async_copy(
        idx_hbm.at[0, pl.ds(0, HEAD * ROWS_PER_W)],
        idx_v.at[pl.ds(0, HEAD * ROWS_PER_W)],
        hsem,
    ).wait()
    for f in range(HEAD):
        gather(f).start()
    pltpu.make_async_copy(
        idx_hbm.at[0, pl.ds(0, (N_FIELDS - HEAD) * ROWS_PER_W)],
        idx_v.at[pl.ds(0, (N_FIELDS - HEAD) * ROWS_PER_W)],
        isem,
    ).wait()
    for f in range(HEAD, N_FIELDS):
        gather(f).start()

    bcopy.wait()
    bias = b_v[...]

    gather(0).wait()

    def init_group(g, carry):
        off = pl.multiple_of(g * 16, 16)
        acc_v[pl.ds(off, 16)] = bias + vals_v[pl.ds(off, 16)]
        return carry

    lax.fori_loop(0, N_GROUPS, init_group, 0)

    for f in range(1, N_FIELDS):
        gather(f).wait()

        def add_group(g, carry, f=f):
            off = pl.multiple_of(g * 16, 16)
            plsc.addupdate(
                acc_v.at[pl.ds(off, 16)],
                vals_v[pl.ds(f * ROWS_PER_W + off, 16)],
            )
            return carry

        lax.fori_loop(0, N_GROUPS, add_group, 0)

    def sig_group(g, carry):
        off = pl.multiple_of(g * 16, 16)
        out_v[pl.ds(off, 16)] = 1.0 / (1.0 + jnp.exp(-acc_v[pl.ds(off, 16)]))
        return carry

    lax.fori_loop(0, N_GROUPS, sig_group, 0)
    pltpu.sync_copy(out_v, out_hbm.at[pl.ds(base, ROWS_PER_W)])


@jax.jit
def kernel(indices, w, b):
    idx_t = indices.astype(jnp.int32).T
    wf = w.T
    b16 = jnp.broadcast_to(b.astype(jnp.float32), (16,))

    run = pl.kernel(
        _sc_body,
        out_type=jax.ShapeDtypeStruct((BATCH,), jnp.float32),
        mesh=plsc.VectorSubcoreMesh(core_axis_name="c", subcore_axis_name="s"),
        compiler_params=pltpu.CompilerParams(needs_layout_passes=False),
        scratch_types=[
            pltpu.VMEM((VALS_PER_W,), jnp.int32),
            pltpu.VMEM((VALS_PER_W,), jnp.float32),
            pltpu.VMEM((ROWS_PER_W,), jnp.float32),
            pltpu.VMEM((ROWS_PER_W,), jnp.float32),
            pltpu.VMEM((16,), jnp.float32),
            pltpu.SemaphoreType.DMA,
            pltpu.SemaphoreType.DMA,
            pltpu.SemaphoreType.DMA((N_FIELDS,)),
            pltpu.SemaphoreType.DMA,
        ],
    )
    return run(idx_t, wf, b16)


# speedup vs baseline: 3.3570x; 1.0114x over previous
import jax
import jax.numpy as jnp
from jax import lax
from jax.experimental import pallas as pl
from jax.experimental.pallas import tpu as pltpu
from jax.experimental.pallas import tpu_sc as plsc

BATCH = 16384
N_FIELDS = 26
INPUT_DIM = 1000000
NC = 2
NS = 16
NW = NC * NS
ROWS_PER_W = BATCH // NW
VALS_PER_W = ROWS_PER_W * N_FIELDS
N_GROUPS = ROWS_PER_W // 16


def _sc_body(idx_hbm, w_hbm, b_hbm, out_hbm, idx_v, vals_v, acc_v, out_v, b_s,
             isem, hsem, gsem, bsem):
    wid = lax.axis_index("s") * NC + lax.axis_index("c")
    base = wid * ROWS_PER_W

    HEAD = 4

    def stage(f, sem):
        return pltpu.make_async_copy(
            idx_hbm.at[f, pl.ds(base, ROWS_PER_W)],
            idx_v.at[pl.ds(f * ROWS_PER_W, ROWS_PER_W)],
            sem,
        )

    for f in range(HEAD):
        stage(f, hsem).start()
    for f in range(HEAD, N_FIELDS):
        stage(f, isem).start()
    bcopy = pltpu.make_async_copy(b_hbm, b_s.at[pl.ds(0, 1)], bsem)
    bcopy.start()

    wrow = w_hbm.at[0]

    def gather(f):
        return pltpu.make_async_copy(
            wrow.at[idx_v.at[pl.ds(f * ROWS_PER_W, ROWS_PER_W)]],
            vals_v.at[pl.ds(f * ROWS_PER_W, ROWS_PER_W)],
            gsem.at[f],
        )

    pltpu.make_async_copy(
        idx_hbm.at[0, pl.ds(0, HEAD * ROWS_PER_W)],
        idx_v.at[pl.ds(0, HEAD * ROWS_PER_W)],
        hsem,
    ).wait()
    for f in range(HEAD):
        gather(f).start()
    pltpu.make_async_copy(
        idx_hbm.at[0, pl.ds(0, (N_FIELDS - HEAD) * ROWS_PER_W)],
        idx_v.at[pl.ds(0, (N_FIELDS - HEAD) * ROWS_PER_W)],
        isem,
    ).wait()
    for f in range(HEAD, N_FIELDS):
        gather(f).start()

    bcopy.wait()
    bias = plsc.load_gather(b_s, [lax.iota(jnp.int32, 16) * 0])

    gather(0).wait()

    def init_group(g, carry):
        off = pl.multiple_of(g * 16, 16)
        acc_v[pl.ds(off, 16)] = bias + vals_v[pl.ds(off, 16)]
        return carry

    lax.fori_loop(0, N_GROUPS, init_group, 0)

    for f in range(1, N_FIELDS):
        gather(f).wait()

        def add_group(g, carry, f=f):
            off = pl.multiple_of(g * 16, 16)
            plsc.addupdate(
                acc_v.at[pl.ds(off, 16)],
                vals_v[pl.ds(f * ROWS_PER_W + off, 16)],
            )
            return carry

        lax.fori_loop(0, N_GROUPS, add_group, 0)

    def sig_group(g, carry):
        off = pl.multiple_of(g * 16, 16)
        out_v[pl.ds(off, 16)] = 1.0 / (1.0 + jnp.exp(-acc_v[pl.ds(off, 16)]))
        return carry

    lax.fori_loop(0, N_GROUPS, sig_group, 0)
    pltpu.sync_copy(out_v, out_hbm.at[pl.ds(base, ROWS_PER_W)])


@jax.jit
def kernel(indices, w, b):
    idx_t = indices.astype(jnp.int32).T
    wf = w.T

    run = pl.kernel(
        _sc_body,
        out_type=jax.ShapeDtypeStruct((BATCH,), jnp.float32),
        mesh=plsc.VectorSubcoreMesh(core_axis_name="c", subcore_axis_name="s"),
        compiler_params=pltpu.CompilerParams(needs_layout_passes=False),
        scratch_types=[
            pltpu.VMEM((VALS_PER_W,), jnp.int32),
            pltpu.VMEM((VALS_PER_W,), jnp.float32),
            pltpu.VMEM((ROWS_PER_W,), jnp.float32),
            pltpu.VMEM((ROWS_PER_W,), jnp.float32),
            pltpu.VMEM((16,), jnp.float32),
            pltpu.SemaphoreType.DMA,
            pltpu.SemaphoreType.DMA,
            pltpu.SemaphoreType.DMA((N_FIELDS,)),
            pltpu.SemaphoreType.DMA,
        ],
    )
    return run(idx_t, wf, b)
